# baseline (speedup 1.0000x reference)
import functools

import jax
import jax.numpy as jnp
from jax import lax
from jax.experimental import pallas as pl
from jax.experimental.pallas import tpu as pltpu
from jax.experimental.pallas import tpu_sc as plsc

NC = 2
NS = 16
NW = NC * NS
LANES = 16
CH = 32


@functools.lru_cache(maxsize=None)
def _build(nb, seq, d):
    pw = seq // NW
    n_h = pw // CH
    n_chunks = n_h * nb
    mesh = plsc.VectorSubcoreMesh(
        core_axis_name="c", subcore_axis_name="s",
        num_cores=NC, num_subcores=NS)

    @functools.partial(
        pl.kernel,
        out_type=jax.ShapeDtypeStruct((nb * seq, d), jnp.float32),
        mesh=mesh,
        scratch_types=[
            pltpu.VMEM((nb * pw,), jnp.int32),
            pltpu.VMEM((2 * CH, d), jnp.float32),
            pltpu.VMEM((CH, d), jnp.float32),
            pltpu.SemaphoreType.DMA,
            pltpu.SemaphoreType.DMA,
            pltpu.SemaphoreType.DMA,
            pltpu.SemaphoreType.DMA,
        ],
    )
    def emb(ids_hbm, wte_hbm, wpe_hbm, out_hbm, idx_all, tok, pos_v,
            sg, ss, si, sp):
        wid = lax.axis_index("s") * NC + lax.axis_index("c")
        pbase = wid * pw

        def gather(k):
            b = lax.rem(k, nb)
            h = k // nb
            pltpu.async_copy(
                wte_hbm.at[idx_all.at[pl.ds(b * pw + h * CH, CH)]],
                tok.at[pl.ds(lax.rem(k, 2) * CH, CH)], sg)

        def drain(sem, ref):
            pltpu.make_async_copy(wpe_hbm.at[pl.ds(0, CH)], ref, sem).wait()

        def add(q):
            base = q * CH

            def body(r, carry):
                for i in range(d // LANES):
                    sl = pl.ds(i * LANES, LANES)
                    tok[base + r, sl] = tok[base + r, sl] + pos_v[r, sl]
                return carry
            lax.fori_loop(0, CH, body, 0)

        def store(k):
            b = lax.rem(k, nb)
            h = k // nb
            pltpu.async_copy(
                tok.at[pl.ds(lax.rem(k, 2) * CH, CH)],
                out_hbm.at[pl.ds(b * seq + pbase + h * CH, CH)], ss)

        def process(k):
            drain(sg, tok.at[pl.ds(0, CH)])
            add(lax.rem(k, 2))
            store(k)

        iws = [pltpu.async_copy(ids_hbm.at[b, pl.ds(pbase, pw)],
                                idx_all.at[pl.ds(b * pw, pw)], si)
               for b in range(nb)]
        pltpu.async_copy(wpe_hbm.at[pl.ds(pbase, CH)], pos_v, sp)
        for iw in iws:
            iw.wait()

        gather(jnp.int32(0))
        gather(jnp.int32(1))
        drain(sp, pos_v)
        process(jnp.int32(0))

        def body(k, carry):
            drain(ss, tok.at[pl.ds(0, CH)])
            gather(k)

            @pl.when(jnp.logical_and(lax.rem(k, nb) == 1, k > nb))
            def _():
                drain(sp, pos_v)

            process(k - 1)

            @pl.when(jnp.logical_and(lax.rem(k, nb) == 0, k >= nb))
            def _():
                pltpu.async_copy(
                    wpe_hbm.at[pl.ds(pbase + (k // nb) * CH, CH)], pos_v, sp)

            return carry

        lax.fori_loop(2, n_chunks, body, 0)

        process(n_chunks - 1)
        drain(ss, tok.at[pl.ds(0, CH)])
        drain(ss, tok.at[pl.ds(0, CH)])

    return emb


def kernel(input_ids, wte, wpe):
    b, s = input_ids.shape
    d = wte.shape[1]
    emb = _build(b, s, d)
    out = emb(input_ids, wte, wpe)
    return out.reshape(b, s, d)


# speedup vs baseline: 1.5065x; 1.5065x over previous
import functools

import jax
import jax.numpy as jnp
from jax import lax
from jax.experimental import pallas as pl
from jax.experimental.pallas import tpu as pltpu
from jax.experimental.pallas import tpu_sc as plsc

NC = 2
NS = 16
NW = NC * NS
LANES = 16
CH = 32
NBUF = 2


@functools.lru_cache(maxsize=None)
def _build(nb, seq, d):
    pw = seq // NW
    n_h = pw // CH
    n_chunks = n_h * nb
    mesh = plsc.VectorSubcoreMesh(
        core_axis_name="c", subcore_axis_name="s",
        num_cores=NC, num_subcores=NS)

    @functools.partial(
        pl.kernel,
        out_type=jax.ShapeDtypeStruct((nb * seq, d), jnp.float32),
        mesh=mesh,
        scratch_types=(
            [pltpu.VMEM((pw,), jnp.int32) for _ in range(nb)]
            + [pltpu.VMEM((CH, d), jnp.float32) for _ in range(NBUF)]
            + [pltpu.VMEM((CH, d), jnp.float32)]
            + [pltpu.SemaphoreType.DMA for _ in range(2 * NBUF + 2)]
        ),
    )
    def emb(ids_hbm, wte_hbm, wpe_hbm, out_hbm, *refs):
        idx = refs[:nb]
        tok = refs[nb:nb + NBUF]
        pos_v = refs[nb + NBUF]
        sg = refs[nb + NBUF + 1:nb + NBUF + 1 + NBUF]
        ss = refs[nb + NBUF + 1 + NBUF:nb + NBUF + 1 + 2 * NBUF]
        si = refs[nb + NBUF + 1 + 2 * NBUF]
        sp = refs[nb + NBUF + 2 + 2 * NBUF]
        wid = lax.axis_index("s") * NC + lax.axis_index("c")
        pbase = wid * pw

        def parts(k):
            return k // nb, k % nb

        def add_half(p, half):
            def body(r, carry):
                for i in range(d // LANES):
                    sl = pl.ds(i * LANES, LANES)
                    tok[p][r, sl] = tok[p][r, sl] + pos_v[r, sl]
                return carry
            lax.fori_loop(half * (CH // 2), (half + 1) * (CH // 2), body, 0)

        def add_store(q, out_off):
            res = []
            for half in range(2):
                add_half(q, half)
                r0 = half * (CH // 2)
                res.append(pltpu.async_copy(
                    tok[q].at[pl.ds(r0, CH // 2)],
                    out_hbm.at[pl.ds(out_off + r0, CH // 2)], ss[q]))
            return res

        def start_gather(k):
            h, b = parts(k)
            return pltpu.async_copy(
                wte_hbm.at[idx[b].at[pl.ds(h * CH, CH)]],
                tok[k % NBUF], sg[k % NBUF])

        g = [None] * NBUF
        s = [None] * NBUF
        pos_pending = [None]

        iws = [pltpu.async_copy(ids_hbm.at[b, pl.ds(pbase, pw)], idx[b], si)
               for b in range(nb)]
        pos_pending[0] = pltpu.async_copy(
            wpe_hbm.at[pl.ds(pbase, CH)], pos_v, sp)
        for iw in iws:
            iw.wait()

        g[0] = start_gather(0)
        for k in range(1, n_chunks):
            p = k % NBUF
            q = 1 - p
            if s[p] is not None:
                for hnd in s[p]:
                    hnd.wait()
            g[p] = start_gather(k)
            if pos_pending[0] is not None:
                pos_pending[0].wait()
                pos_pending[0] = None
            h, b = parts(k - 1)
            g[q].wait()
            s[q] = add_store(q, b * seq + pbase + h * CH)
            if k % nb == 0:
                hh = k // nb
                pos_pending[0] = pltpu.async_copy(
                    wpe_hbm.at[pl.ds(pbase + hh * CH, CH)], pos_v, sp)

        p = (n_chunks - 1) % NBUF
        h, b = parts(n_chunks - 1)
        g[p].wait()
        s[p] = add_store(p, b * seq + pbase + h * CH)
        for hnd in s[1 - p]:
            hnd.wait()
        for hnd in s[p]:
            hnd.wait()

    return emb


def kernel(input_ids, wte, wpe):
    b, s = input_ids.shape
    d = wte.shape[1]
    emb = _build(b, s, d)
    out = emb(input_ids, wte, wpe)
    return out.reshape(b, s, d)


# speedup vs baseline: 1.5259x; 1.0129x over previous
import functools

import jax
import jax.numpy as jnp
from jax import lax
from jax.experimental import pallas as pl
from jax.experimental.pallas import tpu as pltpu
from jax.experimental.pallas import tpu_sc as plsc

NC = 2
NS = 16
NW = NC * NS
LANES = 16
CH = 32
NBUF = 2


@functools.lru_cache(maxsize=None)
def _build(nb, seq, d):
    pw = seq // NW
    n_h = pw // CH
    n_chunks = n_h * nb
    mesh = plsc.VectorSubcoreMesh(
        core_axis_name="c", subcore_axis_name="s",
        num_cores=NC, num_subcores=NS)

    @functools.partial(
        pl.kernel,
        out_type=jax.ShapeDtypeStruct((nb * seq, d), jnp.float32),
        mesh=mesh,
        scratch_types=(
            [pltpu.VMEM((pw,), jnp.int32) for _ in range(nb)]
            + [pltpu.VMEM((CH, d), jnp.float32) for _ in range(NBUF)]
            + [pltpu.VMEM((CH, d), jnp.float32)]
            + [pltpu.SemaphoreType.DMA for _ in range(2 * NBUF + 2)]
        ),
    )
    def emb(ids_hbm, wte_hbm, wpe_hbm, out_hbm, *refs):
        idx = refs[:nb]
        tok = refs[nb:nb + NBUF]
        pos_v = refs[nb + NBUF]
        sg = refs[nb + NBUF + 1:nb + NBUF + 1 + NBUF]
        ss = refs[nb + NBUF + 1 + NBUF:nb + NBUF + 1 + 2 * NBUF]
        si = refs[nb + NBUF + 1 + 2 * NBUF]
        sp = refs[nb + NBUF + 2 + 2 * NBUF]
        wid = lax.axis_index("s") * NC + lax.axis_index("c")
        pbase = wid * pw

        def parts(k):
            return k // nb, k % nb

        def add_half(p, half):
            def body(r, carry):
                for i in range(d // LANES):
                    sl = pl.ds(i * LANES, LANES)
                    tok[p][r, sl] = tok[p][r, sl] + pos_v[r, sl]
                return carry
            lax.fori_loop(half * (CH // 2), (half + 1) * (CH // 2), body, 0)

        def add_store(q, out_off):
            res = []
            for half in range(2):
                g[q][half].wait()
                add_half(q, half)
                r0 = half * (CH // 2)
                res.append(pltpu.async_copy(
                    tok[q].at[pl.ds(r0, CH // 2)],
                    out_hbm.at[pl.ds(out_off + r0, CH // 2)], ss[q]))
            return res

        def start_gather(k):
            h, b = parts(k)
            p = k % NBUF
            return [
                pltpu.async_copy(
                    wte_hbm.at[idx[b].at[pl.ds(h * CH + half * (CH // 2),
                                               CH // 2)]],
                    tok[p].at[pl.ds(half * (CH // 2), CH // 2)], sg[p])
                for half in range(2)
            ]

        g = [None] * NBUF
        s = [None] * NBUF
        pos_pending = [None]

        iws = [pltpu.async_copy(ids_hbm.at[b, pl.ds(pbase, pw)], idx[b], si)
               for b in range(nb)]
        pos_pending[0] = pltpu.async_copy(
            wpe_hbm.at[pl.ds(pbase, CH)], pos_v, sp)
        for iw in iws:
            iw.wait()

        g[0] = start_gather(0)
        for k in range(1, n_chunks):
            p = k % NBUF
            q = 1 - p
            if s[p] is not None:
                for hnd in s[p]:
                    hnd.wait()
            g[p] = start_gather(k)
            if pos_pending[0] is not None:
                pos_pending[0].wait()
                pos_pending[0] = None
            h, b = parts(k - 1)
            s[q] = add_store(q, b * seq + pbase + h * CH)
            if k % nb == 0:
                hh = k // nb
                pos_pending[0] = pltpu.async_copy(
                    wpe_hbm.at[pl.ds(pbase + hh * CH, CH)], pos_v, sp)

        p = (n_chunks - 1) % NBUF
        h, b = parts(n_chunks - 1)
        s[p] = add_store(p, b * seq + pbase + h * CH)
        for hnd in s[1 - p]:
            hnd.wait()
        for hnd in s[p]:
            hnd.wait()

    return emb


def kernel(input_ids, wte, wpe):
    b, s = input_ids.shape
    d = wte.shape[1]
    emb = _build(b, s, d)
    out = emb(input_ids, wte, wpe)
    return out.reshape(b, s, d)
